# Initial kernel scaffold; baseline (speedup 1.0000x reference)
import jax
import jax.numpy as jnp
from jax.experimental import pallas as pl


def kernel(x_ncl, conv_w, conv_b, bn_gamma, bn_beta):
    raise NotImplementedError("write your pallas kernel here")


# speedup vs baseline: 1.3974x; 1.3974x over previous
import functools

import jax
import jax.numpy as jnp
from jax.experimental import pallas as pl
from jax.experimental.pallas import tpu as pltpu

EPS = 1e-5


def _stats_kernel(x_ref, w_ref, b_ref, sum_ref, sumsq_ref):
    @pl.when(pl.program_id(1) == 0)
    def _():
        sum_ref[...] = jnp.zeros_like(sum_ref)
        sumsq_ref[...] = jnp.zeros_like(sumsq_ref)

    x = x_ref[0]
    w = w_ref[...]
    y = jnp.dot(w, x, preferred_element_type=jnp.float32) + b_ref[...]
    y = jnp.maximum(y, 0.0)
    sum_ref[0] += jnp.sum(y, axis=1, keepdims=True)
    sumsq_ref[0] += jnp.sum(y * y, axis=1, keepdims=True)


def _apply_kernel(x_ref, w_ref, b_ref, g_ref, be_ref, s_ref, ss_ref, o_ref,
                  *, inv_count):
    s = jnp.sum(s_ref[...], axis=0)
    ss = jnp.sum(ss_ref[...], axis=0)
    mean = s * inv_count
    var = jnp.maximum(ss * inv_count - mean * mean, 0.0)
    scale = g_ref[...] * jax.lax.rsqrt(var + EPS)
    shift = be_ref[...] - mean * scale

    x = x_ref[0]
    w = w_ref[...]
    y = jnp.dot(w, x, preferred_element_type=jnp.float32) + b_ref[...]
    y = jnp.maximum(y, 0.0)
    o_ref[0] = (y * scale + shift).astype(o_ref.dtype)


def kernel(x_ncl, conv_w, conv_b, bn_gamma, bn_beta):
    N, C_in, L = x_ncl.shape
    C_out = conv_w.shape[0]

    w = conv_w[:, :, 0]
    b = conv_b.reshape(C_out, 1).astype(jnp.float32)
    g = bn_gamma.reshape(C_out, 1).astype(jnp.float32)
    be = bn_beta.reshape(C_out, 1).astype(jnp.float32)

    G = 2 if N % 2 == 0 else 1
    per = N // G

    w_spec = pl.BlockSpec((C_out, C_in), lambda *_: (0, 0))

    def vec_spec():
        return pl.BlockSpec((C_out, 1), lambda *_: (0, 0))

    sums, sumsqs = pl.pallas_call(
        _stats_kernel,
        grid=(G, per),
        in_specs=[
            pl.BlockSpec((1, C_in, L), lambda c, i: (c * per + i, 0, 0)),
            w_spec,
            vec_spec(),
        ],
        out_specs=[pl.BlockSpec((1, C_out, 1), lambda c, i: (c, 0, 0))] * 2,
        out_shape=[jax.ShapeDtypeStruct((G, C_out, 1), jnp.float32)] * 2,
        compiler_params=pltpu.CompilerParams(
            dimension_semantics=("parallel", "arbitrary")),
    )(x_ncl, w, b)

    out = pl.pallas_call(
        functools.partial(_apply_kernel, inv_count=1.0 / float(N * L)),
        grid=(N,),
        in_specs=[
            pl.BlockSpec((1, C_in, L), lambda n: (n, 0, 0)),
            w_spec,
            vec_spec(),
            vec_spec(),
            vec_spec(),
            pl.BlockSpec((G, C_out, 1), lambda n: (0, 0, 0)),
            pl.BlockSpec((G, C_out, 1), lambda n: (0, 0, 0)),
        ],
        out_specs=pl.BlockSpec((1, C_out, L), lambda n: (n, 0, 0)),
        out_shape=jax.ShapeDtypeStruct((N, C_out, L), x_ncl.dtype),
        compiler_params=pltpu.CompilerParams(
            dimension_semantics=("parallel",)),
    )(x_ncl, w, b, g, be, sums, sumsqs)
    return out


# speedup vs baseline: 1.4067x; 1.0066x over previous
import functools

import jax
import jax.numpy as jnp
from jax.experimental import pallas as pl
from jax.experimental.pallas import tpu as pltpu

EPS = 1e-5


def _stats_kernel(x_ref, w_ref, b_ref, sum_ref, sumsq_ref):
    @pl.when(pl.program_id(1) == 0)
    def _():
        sum_ref[...] = jnp.zeros_like(sum_ref)
        sumsq_ref[...] = jnp.zeros_like(sumsq_ref)

    x = x_ref[0].astype(jnp.bfloat16)
    w = w_ref[...]
    y = jnp.dot(w, x, preferred_element_type=jnp.float32) + b_ref[...]
    y = jnp.maximum(y, 0.0)
    sum_ref[0] += jnp.sum(y, axis=1, keepdims=True)
    sumsq_ref[0] += jnp.sum(y * y, axis=1, keepdims=True)


def _apply_kernel(x_ref, w_ref, b_ref, g_ref, be_ref, s_ref, ss_ref, o_ref,
                  *, inv_count):
    s = jnp.sum(s_ref[...], axis=0)
    ss = jnp.sum(ss_ref[...], axis=0)
    mean = s * inv_count
    var = jnp.maximum(ss * inv_count - mean * mean, 0.0)
    scale = g_ref[...] * jax.lax.rsqrt(var + EPS)
    shift = be_ref[...] - mean * scale

    x = x_ref[0].astype(jnp.bfloat16)
    w = w_ref[...]
    y = jnp.dot(w, x, preferred_element_type=jnp.float32) + b_ref[...]
    y = jnp.maximum(y, 0.0)
    o_ref[0] = (y * scale + shift).astype(o_ref.dtype)


def kernel(x_ncl, conv_w, conv_b, bn_gamma, bn_beta):
    N, C_in, L = x_ncl.shape
    C_out = conv_w.shape[0]

    w = conv_w[:, :, 0].astype(jnp.bfloat16)
    b = conv_b.reshape(C_out, 1).astype(jnp.float32)
    g = bn_gamma.reshape(C_out, 1).astype(jnp.float32)
    be = bn_beta.reshape(C_out, 1).astype(jnp.float32)

    G = 2 if N % 2 == 0 else 1
    per = N // G

    w_spec = pl.BlockSpec((C_out, C_in), lambda *_: (0, 0))

    def vec_spec():
        return pl.BlockSpec((C_out, 1), lambda *_: (0, 0))

    sums, sumsqs = pl.pallas_call(
        _stats_kernel,
        grid=(G, per),
        in_specs=[
            pl.BlockSpec((1, C_in, L), lambda c, i: (c * per + i, 0, 0)),
            w_spec,
            vec_spec(),
        ],
        out_specs=[pl.BlockSpec((1, C_out, 1), lambda c, i: (c, 0, 0))] * 2,
        out_shape=[jax.ShapeDtypeStruct((G, C_out, 1), jnp.float32)] * 2,
        compiler_params=pltpu.CompilerParams(
            dimension_semantics=("parallel", "arbitrary")),
    )(x_ncl, w, b)

    out = pl.pallas_call(
        functools.partial(_apply_kernel, inv_count=1.0 / float(N * L)),
        grid=(N,),
        in_specs=[
            pl.BlockSpec((1, C_in, L), lambda n: (n, 0, 0)),
            w_spec,
            vec_spec(),
            vec_spec(),
            vec_spec(),
            pl.BlockSpec((G, C_out, 1), lambda n: (0, 0, 0)),
            pl.BlockSpec((G, C_out, 1), lambda n: (0, 0, 0)),
        ],
        out_specs=pl.BlockSpec((1, C_out, L), lambda n: (n, 0, 0)),
        out_shape=jax.ShapeDtypeStruct((N, C_out, L), x_ncl.dtype),
        compiler_params=pltpu.CompilerParams(
            dimension_semantics=("parallel",)),
    )(x_ncl, w, b, g, be, sums, sumsqs)
    return out
